# baseline (device time: 20255 ns/iter reference)
import jax
import jax.numpy as jnp
from jax import lax
from jax.experimental import pallas as pl
from jax.experimental.pallas import tpu as pltpu

N_DEV = 4


def kernel(A, B):
    m, _ = A.shape
    _, n = B.shape
    m_out = m // N_DEV

    def body(a_ref, b_ref, out_ref, send_ref, recv_ref, send_sems, recv_sems):
        p = lax.axis_index("i")
        left = (p - 1) % N_DEV
        right = (p + 1) % N_DEV

        barrier = pltpu.get_barrier_semaphore()
        for nbr in (left, right):
            pl.semaphore_signal(
                barrier, inc=1,
                device_id=(nbr,), device_id_type=pl.DeviceIdType.MESH,
            )
        pl.semaphore_wait(barrier, 2)

        b = b_ref[:, :].astype(jnp.bfloat16)

        def chunk_partial(c):
            a = a_ref[pl.ds(c * m_out, m_out), :].astype(jnp.bfloat16)
            return jnp.dot(a, b, preferred_element_type=jnp.float32)

        for h in range(N_DEV - 1):
            c_send = (p - h - 1) % N_DEV
            part = chunk_partial(c_send)
            if h == 0:
                send_ref[0, :, :] = part
            else:
                send_ref[h % 2, :, :] = recv_ref[h - 1, :, :] + part
            rdma = pltpu.make_async_remote_copy(
                src_ref=send_ref.at[h % 2],
                dst_ref=recv_ref.at[h],
                send_sem=send_sems.at[h],
                recv_sem=recv_sems.at[h],
                device_id=(right,),
                device_id_type=pl.DeviceIdType.MESH,
            )
            rdma.start()
            rdma.wait()

        out_ref[:, :] = recv_ref[N_DEV - 2, :, :] + chunk_partial(p)

    return pl.pallas_call(
        body,
        out_shape=jax.ShapeDtypeStruct((m_out, n), jnp.float32),
        in_specs=[
            pl.BlockSpec(memory_space=pltpu.VMEM),
            pl.BlockSpec(memory_space=pltpu.VMEM),
        ],
        out_specs=pl.BlockSpec(memory_space=pltpu.VMEM),
        scratch_shapes=[
            pltpu.VMEM((2, m_out, n), jnp.float32),
            pltpu.VMEM((N_DEV - 1, m_out, n), jnp.float32),
            pltpu.SemaphoreType.DMA((N_DEV - 1,)),
            pltpu.SemaphoreType.DMA((N_DEV - 1,)),
        ],
        compiler_params=pltpu.CompilerParams(collective_id=0),
    )(A, B)


# device time: 11179 ns/iter; 1.8119x vs baseline; 1.8119x over previous
import jax
import jax.numpy as jnp
from jax import lax
from jax.experimental import pallas as pl
from jax.experimental.pallas import tpu as pltpu

N_DEV = 4


def kernel(A, B):
    m, _ = A.shape
    _, n = B.shape
    m_out = m // N_DEV

    def body(a_ref, b_ref, out_ref, send_ref, recv_ref, send_sems, recv_sems):
        p = lax.axis_index("i")

        barrier = pltpu.get_barrier_semaphore()
        for d in range(1, N_DEV):
            pl.semaphore_signal(
                barrier, inc=1,
                device_id=((p + d) % N_DEV,),
                device_id_type=pl.DeviceIdType.MESH,
            )
        pl.semaphore_wait(barrier, N_DEV - 1)

        b = b_ref[:, :].astype(jnp.bfloat16)

        def chunk_partial(c):
            a = a_ref[pl.ds(c * m_out, m_out), :].astype(jnp.bfloat16)
            return jnp.dot(a, b, preferred_element_type=jnp.float32)

        rdmas = []
        for d in range(1, N_DEV):
            q = (p + d) % N_DEV
            send_ref[d - 1, :, :] = chunk_partial(q).astype(jnp.bfloat16)
            rdma = pltpu.make_async_remote_copy(
                src_ref=send_ref.at[d - 1],
                dst_ref=recv_ref.at[N_DEV - 1 - d],
                send_sem=send_sems.at[d - 1],
                recv_sem=recv_sems.at[N_DEV - 1 - d],
                device_id=(q,),
                device_id_type=pl.DeviceIdType.MESH,
            )
            rdma.start()
            rdmas.append(rdma)

        acc = chunk_partial(p)

        for rdma in rdmas:
            rdma.wait()
        out_ref[:, :] = (
            acc
            + recv_ref[0, :, :].astype(jnp.float32)
            + recv_ref[1, :, :].astype(jnp.float32)
            + recv_ref[2, :, :].astype(jnp.float32)
        )

    return pl.pallas_call(
        body,
        out_shape=jax.ShapeDtypeStruct((m_out, n), jnp.float32),
        in_specs=[
            pl.BlockSpec(memory_space=pltpu.VMEM),
            pl.BlockSpec(memory_space=pltpu.VMEM),
        ],
        out_specs=pl.BlockSpec(memory_space=pltpu.VMEM),
        scratch_shapes=[
            pltpu.VMEM((N_DEV - 1, m_out, n), jnp.bfloat16),
            pltpu.VMEM((N_DEV - 1, m_out, n), jnp.bfloat16),
            pltpu.SemaphoreType.DMA((N_DEV - 1,)),
            pltpu.SemaphoreType.DMA((N_DEV - 1,)),
        ],
        compiler_params=pltpu.CompilerParams(collective_id=0),
    )(A, B)


# device time: 11023 ns/iter; 1.8375x vs baseline; 1.0142x over previous
import jax
import jax.numpy as jnp
from jax import lax
from jax.experimental import pallas as pl
from jax.experimental.pallas import tpu as pltpu

N_DEV = 4


def kernel(A, B):
    m, _ = A.shape
    _, n = B.shape
    m_out = m // N_DEV

    def body(a_ref, b_ref, out_ref, part_ref, send_ref, recv_ref,
             send_sems, recv_sems):
        p = lax.axis_index("i")

        barrier = pltpu.get_barrier_semaphore()
        for d in range(1, N_DEV):
            pl.semaphore_signal(
                barrier, inc=1,
                device_id=((p + d) % N_DEV,),
                device_id_type=pl.DeviceIdType.MESH,
            )
        pl.semaphore_wait(barrier, N_DEV - 1)

        part_ref[:, :] = jnp.dot(
            a_ref[:, :].astype(jnp.bfloat16),
            b_ref[:, :].astype(jnp.bfloat16),
            preferred_element_type=jnp.float32,
        )

        def chunk_partial(c):
            return part_ref[pl.ds(c * m_out, m_out), :]

        rdmas = []
        for d in (2, 1, 3):
            q = (p + d) % N_DEV
            send_ref[d - 1, :, :] = chunk_partial(q).astype(jnp.bfloat16)
            rdma = pltpu.make_async_remote_copy(
                src_ref=send_ref.at[d - 1],
                dst_ref=recv_ref.at[N_DEV - 1 - d],
                send_sem=send_sems.at[d - 1],
                recv_sem=recv_sems.at[N_DEV - 1 - d],
                device_id=(q,),
                device_id_type=pl.DeviceIdType.MESH,
            )
            rdma.start()
            rdmas.append(rdma)

        acc = chunk_partial(p)

        for rdma in rdmas:
            rdma.wait()
        out_ref[:, :] = (
            acc
            + recv_ref[0, :, :].astype(jnp.float32)
            + recv_ref[1, :, :].astype(jnp.float32)
            + recv_ref[2, :, :].astype(jnp.float32)
        )

    return pl.pallas_call(
        body,
        out_shape=jax.ShapeDtypeStruct((m_out, n), jnp.float32),
        in_specs=[
            pl.BlockSpec(memory_space=pltpu.VMEM),
            pl.BlockSpec(memory_space=pltpu.VMEM),
        ],
        out_specs=pl.BlockSpec(memory_space=pltpu.VMEM),
        scratch_shapes=[
            pltpu.VMEM((m, n), jnp.float32),
            pltpu.VMEM((N_DEV - 1, m_out, n), jnp.bfloat16),
            pltpu.VMEM((N_DEV - 1, m_out, n), jnp.bfloat16),
            pltpu.SemaphoreType.DMA((N_DEV - 1,)),
            pltpu.SemaphoreType.DMA((N_DEV - 1,)),
        ],
        compiler_params=pltpu.CompilerParams(collective_id=0),
    )(A, B)
